# baseline (speedup 1.0000x reference)
import jax
import jax.numpy as jnp
from jax import lax
from jax.experimental import pallas as pl
from jax.experimental.pallas import tpu as pltpu
from jax.experimental.pallas import tpu_sc as plsc

N = 10000
D = 128
E = 320000
NC = 2
NS = 16
NW = NC * NS
EPW = 10240
EPAD = NW * EPW - E
CHUNK = 128
NCHUNK = EPW // CHUNK
GRP = 5
NGRP = NCHUNK // GRP
NPAD = 10240
RPS = NPAD // NS
ZROWS = 32
BR = 2000

_MESH = plsc.VectorSubcoreMesh(
    core_axis_name="c", subcore_axis_name="s", num_cores=NC, num_subcores=NS)


def _sc_scatter_body(g_hbm, src_hbm, dst_hbm, dep_hbm, p_hbm,
                     src_v, dst_v, rows_a, rows_b, acc_sh,
                     gsem_a, gsem_b, ssem_a, ssem_b):
  c = lax.axis_index("c")
  s = lax.axis_index("s")
  wid = c * NS + s

  def wait_chunk(buf, sem):
    pltpu.make_async_copy(g_hbm.at[pl.ds(0, CHUNK)], buf, sem).wait()

  def gather(cc, buf, sem):
    pltpu.async_copy(g_hbm.at[src_v.at[(cc // GRP) % 2, cc % GRP]], buf, sem)

  def scatter(cc, buf, sem):
    pltpu.async_copy(buf, acc_sh.at[dst_v.at[(cc // GRP) % 2, cc % GRP]],
                     sem, add=True)

  def maybe_reload(cc):
    @pl.when(cc % GRP == 0)
    def _():
      g = cc // GRP
      pltpu.sync_copy(src_hbm.at[wid, g], src_v.at[g % 2])
      pltpu.sync_copy(dst_hbm.at[wid, g], dst_v.at[g % 2])

  zero16 = jnp.zeros((16,), jnp.float32)

  @pl.loop(0, CHUNK)
  def _(i):
    @pl.loop(0, D // 16)
    def _(j):
      rows_a[i, pl.ds(j * 16, 16)] = zero16

  @pl.loop(0, RPS // CHUNK)
  def _(i):
    pltpu.sync_copy(rows_a, acc_sh.at[pl.ds(s * RPS + i * CHUNK, CHUNK)])

  pltpu.sync_copy(src_hbm.at[wid, 0], src_v.at[0])
  pltpu.sync_copy(dst_hbm.at[wid, 0], dst_v.at[0])
  plsc.subcore_barrier()

  gather(0, rows_a, gsem_a)
  gather(1, rows_b, gsem_b)

  @pl.loop(0, (NCHUNK - 2) // 2)
  def _(i):
    c0 = 2 * i
    wait_chunk(rows_a, gsem_a)
    scatter(c0, rows_a, ssem_a)
    wait_chunk(rows_b, gsem_b)
    scatter(c0 + 1, rows_b, ssem_b)

    maybe_reload(c0 + 2)
    wait_chunk(rows_a, ssem_a)
    gather(c0 + 2, rows_a, gsem_a)
    maybe_reload(c0 + 3)
    wait_chunk(rows_b, ssem_b)
    gather(c0 + 3, rows_b, gsem_b)

  wait_chunk(rows_a, gsem_a)
  scatter(NCHUNK - 2, rows_a, ssem_a)
  wait_chunk(rows_b, gsem_b)
  scatter(NCHUNK - 1, rows_b, ssem_b)
  wait_chunk(rows_a, ssem_a)
  wait_chunk(rows_b, ssem_b)

  plsc.subcore_barrier()

  @pl.loop(0, RPS // CHUNK)
  def _(i):
    pltpu.sync_copy(acc_sh.at[pl.ds(s * RPS + i * CHUNK, CHUNK)], rows_a)
    pltpu.sync_copy(rows_a, p_hbm.at[c].at[pl.ds(s * RPS + i * CHUNK, CHUNK)])


_sc_scatter = pl.kernel(
    _sc_scatter_body,
    out_type=jax.ShapeDtypeStruct((NC, NPAD, D), jnp.float32),
    mesh=_MESH,
    scratch_types=[
        pltpu.VMEM((2, GRP, CHUNK), jnp.int32),
        pltpu.VMEM((2, GRP, CHUNK), jnp.int32),
        pltpu.VMEM((CHUNK, D), jnp.float32),
        pltpu.VMEM((CHUNK, D), jnp.float32),
        pltpu.VMEM_SHARED((NPAD, D), jnp.float32),
        pltpu.SemaphoreType.DMA,
        pltpu.SemaphoreType.DMA,
        pltpu.SemaphoreType.DMA,
        pltpu.SemaphoreType.DMA,
    ])


def _sc_deg_body(dst_hbm, deg_hbm, dst_v, ones_v, zbuf, acc_sh):
  c = lax.axis_index("c")
  s = lax.axis_index("s")
  wid = c * NS + s
  zero16 = jnp.zeros((16,), jnp.float32)
  one16 = jnp.ones((16,), jnp.float32)

  @pl.loop(0, ZROWS)
  def _(i):
    @pl.loop(0, D // 16)
    def _(j):
      zbuf[i, pl.ds(j * 16, 16)] = zero16

  @pl.loop(0, CHUNK)
  def _(i):
    @pl.loop(0, D // 16)
    def _(j):
      ones_v[i, pl.ds(j * 16, 16)] = one16

  @pl.loop(0, RPS // ZROWS)
  def _(i):
    pltpu.sync_copy(zbuf, acc_sh.at[pl.ds(s * RPS + i * ZROWS, ZROWS)])

  pltpu.sync_copy(dst_hbm.at[wid], dst_v)
  plsc.subcore_barrier()

  @pl.loop(0, NCHUNK)
  def _(j):
    pltpu.sync_copy(ones_v, acc_sh.at[dst_v.at[j // GRP, j % GRP]], add=True)

  plsc.subcore_barrier()

  @pl.loop(0, RPS // ZROWS)
  def _(i):
    pltpu.sync_copy(acc_sh.at[pl.ds(s * RPS + i * ZROWS, ZROWS)], zbuf)
    pltpu.sync_copy(zbuf, deg_hbm.at[c].at[pl.ds(s * RPS + i * ZROWS, ZROWS)])


_sc_deg = pl.kernel(
    _sc_deg_body,
    out_type=jax.ShapeDtypeStruct((NC, NPAD, D), jnp.float32),
    mesh=_MESH,
    scratch_types=[
        pltpu.VMEM((NGRP, GRP, CHUNK), jnp.int32),
        pltpu.VMEM((CHUNK, D), jnp.float32),
        pltpu.VMEM((ZROWS, D), jnp.float32),
        pltpu.VMEM_SHARED((NPAD, D), jnp.float32),
    ])


def _tc_pre(x, w_self, w_neigh, b):
  def body(x_ref, ws_ref, wn_ref, b_ref, s_ref, g_ref):
    h = x_ref[...]
    s_ref[...] = jnp.dot(h, ws_ref[...],
                         preferred_element_type=jnp.float32) + b_ref[...]
    g_ref[...] = jnp.dot(h, wn_ref[...], preferred_element_type=jnp.float32)

  return pl.pallas_call(
      body,
      grid=(N // BR,),
      in_specs=[pl.BlockSpec((BR, D), lambda i: (i, 0)),
                pl.BlockSpec((D, D), lambda i: (0, 0)),
                pl.BlockSpec((D, D), lambda i: (0, 0)),
                pl.BlockSpec((1, D), lambda i: (0, 0))],
      out_specs=[pl.BlockSpec((BR, D), lambda i: (i, 0)),
                 pl.BlockSpec((BR, D), lambda i: (i, 0))],
      out_shape=[jax.ShapeDtypeStruct((N, D), jnp.float32)] * 2,
  )(x, w_self, w_neigh, b.reshape(1, D))


def _combine(s_ref, p_ref, deg_ref):
  p = p_ref[0] + p_ref[1]
  deg = deg_ref[0, :, 0:1] + deg_ref[1, :, 0:1]
  inv = 1.0 / jnp.maximum(deg, 1.0)
  return s_ref[...] + p * inv


def _tc_mid(s_in, p, degp, w_self, w_neigh, b):
  def body(s_ref, p_ref, deg_ref, ws_ref, wn_ref, b_ref, s_ref_o, g_ref_o):
    h = jnp.maximum(_combine(s_ref, p_ref, deg_ref), 0.0)
    s_ref_o[...] = jnp.dot(h, ws_ref[...],
                           preferred_element_type=jnp.float32) + b_ref[...]
    g_ref_o[...] = jnp.dot(h, wn_ref[...], preferred_element_type=jnp.float32)

  return pl.pallas_call(
      body,
      grid=(N // BR,),
      in_specs=[pl.BlockSpec((BR, D), lambda i: (i, 0)),
                pl.BlockSpec((NC, BR, D), lambda i: (0, i, 0)),
                pl.BlockSpec((NC, BR, D), lambda i: (0, i, 0)),
                pl.BlockSpec((D, D), lambda i: (0, 0)),
                pl.BlockSpec((D, D), lambda i: (0, 0)),
                pl.BlockSpec((1, D), lambda i: (0, 0))],
      out_specs=[pl.BlockSpec((BR, D), lambda i: (i, 0)),
                 pl.BlockSpec((BR, D), lambda i: (i, 0))],
      out_shape=[jax.ShapeDtypeStruct((N, D), jnp.float32)] * 2,
  )(s_in, p, degp, w_self, w_neigh, b.reshape(1, D))


def _tc_final(s_in, p, degp):
  def body(s_ref, p_ref, deg_ref, o_ref):
    o_ref[...] = _combine(s_ref, p_ref, deg_ref)

  return pl.pallas_call(
      body,
      grid=(N // BR,),
      in_specs=[pl.BlockSpec((BR, D), lambda i: (i, 0)),
                pl.BlockSpec((NC, BR, D), lambda i: (0, i, 0)),
                pl.BlockSpec((NC, BR, D), lambda i: (0, i, 0))],
      out_specs=pl.BlockSpec((BR, D), lambda i: (i, 0)),
      out_shape=jax.ShapeDtypeStruct((N, D), jnp.float32),
  )(s_in, p, degp)


def kernel(x, edge_index,
           W_self0, W_neigh0, b0,
           W_self1, W_neigh1, b1,
           W_self2, W_neigh2, b2,
           W_self3, W_neigh3, b3):
  ei = edge_index.astype(jnp.int32)
  src = jnp.concatenate([ei[0], jnp.zeros((EPAD,), jnp.int32)])
  dst = jnp.concatenate([ei[1], jnp.full((EPAD,), N, jnp.int32)])
  src = src.reshape(NW, NGRP, GRP, CHUNK)
  dst = dst.reshape(NW, NGRP, GRP, CHUNK)

  degp = _sc_deg(dst)
  dep = degp[0, :8]
  s0, g0 = _tc_pre(x, W_self0, W_neigh0, b0)
  p0 = _sc_scatter(g0, src, dst, dep)
  s1, g1 = _tc_mid(s0, p0, degp, W_self1, W_neigh1, b1)
  p1 = _sc_scatter(g1, src, dst, dep)
  s2, g2 = _tc_mid(s1, p1, degp, W_self2, W_neigh2, b2)
  p2 = _sc_scatter(g2, src, dst, dep)
  s3, g3 = _tc_mid(s2, p2, degp, W_self3, W_neigh3, b3)
  p3 = _sc_scatter(g3, src, dst, dep)
  return _tc_final(s3, p3, degp)


# speedup vs baseline: 3.2386x; 3.2386x over previous
import jax
import jax.numpy as jnp
from jax import lax
from jax.experimental import pallas as pl
from jax.experimental.pallas import tpu as pltpu
from jax.experimental.pallas import tpu_sc as plsc

N = 10000
D = 128
E = 320000
NC = 2
NS = 16
NW = NC * NS
EPW = 10240
EPAD = NW * EPW - E
CHUNK = 128
NCHUNK = EPW // CHUNK
GRP = 5
NGRP = NCHUNK // GRP
NPAD = 10240
RPS = NPAD // NS
ZROWS = 32
BR = 2000

_MESH = plsc.VectorSubcoreMesh(
    core_axis_name="c", subcore_axis_name="s", num_cores=NC, num_subcores=NS)


def _sc_scatter_body(g_hbm, src_hbm, dst_hbm, dep_hbm, p_hbm,
                     src_v, dst_v, rows_a, rows_b, acc_sh,
                     gsem_a, gsem_b, ssem_a, ssem_b):
  c = lax.axis_index("c")
  s = lax.axis_index("s")
  wid = c * NS + s

  def wait_chunk(buf, sem):
    pltpu.make_async_copy(g_hbm.at[pl.ds(0, CHUNK)], buf, sem).wait()

  def gather(cc, buf, sem):
    pltpu.async_copy(g_hbm.at[src_v.at[(cc // GRP) % 2, cc % GRP]], buf, sem)

  def scatter(cc, buf, sem):
    pltpu.async_copy(buf, acc_sh.at[dst_v.at[(cc // GRP) % 2, cc % GRP]],
                     sem, add=True)

  def maybe_reload(cc):
    @pl.when(cc % GRP == 0)
    def _():
      g = cc // GRP
      pltpu.sync_copy(src_hbm.at[wid, g], src_v.at[g % 2])
      pltpu.sync_copy(dst_hbm.at[wid, g], dst_v.at[g % 2])

  zero16 = jnp.zeros((16,), jnp.float32)

  @pl.loop(0, CHUNK)
  def _(i):
    @pl.loop(0, D // 16)
    def _(j):
      rows_a[i, pl.ds(j * 16, 16)] = zero16

  @pl.loop(0, RPS // CHUNK)
  def _(i):
    pltpu.sync_copy(rows_a, acc_sh.at[pl.ds(s * RPS + i * CHUNK, CHUNK)])

  pltpu.sync_copy(src_hbm.at[wid, 0], src_v.at[0])
  pltpu.sync_copy(dst_hbm.at[wid, 0], dst_v.at[0])
  plsc.subcore_barrier()

  gather(0, rows_a, gsem_a)
  gather(1, rows_b, gsem_b)

  @pl.loop(0, (NCHUNK - 2) // 2)
  def _(i):
    c0 = 2 * i
    wait_chunk(rows_a, gsem_a)
    scatter(c0, rows_a, ssem_a)
    wait_chunk(rows_b, gsem_b)
    scatter(c0 + 1, rows_b, ssem_b)

    maybe_reload(c0 + 2)
    wait_chunk(rows_a, ssem_a)
    gather(c0 + 2, rows_a, gsem_a)
    maybe_reload(c0 + 3)
    wait_chunk(rows_b, ssem_b)
    gather(c0 + 3, rows_b, gsem_b)

  wait_chunk(rows_a, gsem_a)
  scatter(NCHUNK - 2, rows_a, ssem_a)
  wait_chunk(rows_b, gsem_b)
  scatter(NCHUNK - 1, rows_b, ssem_b)
  wait_chunk(rows_a, ssem_a)
  wait_chunk(rows_b, ssem_b)

  plsc.subcore_barrier()

  @pl.loop(0, RPS // CHUNK)
  def _(i):
    pltpu.sync_copy(acc_sh.at[pl.ds(s * RPS + i * CHUNK, CHUNK)], rows_a)
    pltpu.sync_copy(rows_a, p_hbm.at[c].at[pl.ds(s * RPS + i * CHUNK, CHUNK)])


_sc_scatter = pl.kernel(
    _sc_scatter_body,
    out_type=jax.ShapeDtypeStruct((NC, NPAD, D), jnp.float32),
    mesh=_MESH,
    scratch_types=[
        pltpu.VMEM((2, GRP, CHUNK), jnp.int32),
        pltpu.VMEM((2, GRP, CHUNK), jnp.int32),
        pltpu.VMEM((CHUNK, D), jnp.float32),
        pltpu.VMEM((CHUNK, D), jnp.float32),
        pltpu.VMEM_SHARED((NPAD, D), jnp.float32),
        pltpu.SemaphoreType.DMA,
        pltpu.SemaphoreType.DMA,
        pltpu.SemaphoreType.DMA,
        pltpu.SemaphoreType.DMA,
    ])


def _sc_deg_body(dst_hbm, deg_hbm, dst_v, ones_v, zbuf, acc_sh):
  c = lax.axis_index("c")
  s = lax.axis_index("s")
  wid = c * NS + s
  zero16 = jnp.zeros((16,), jnp.float32)
  one16 = jnp.ones((16,), jnp.float32)

  @pl.loop(0, ZROWS)
  def _(i):
    @pl.loop(0, D // 16)
    def _(j):
      zbuf[i, pl.ds(j * 16, 16)] = zero16

  @pl.loop(0, CHUNK)
  def _(i):
    @pl.loop(0, D // 16)
    def _(j):
      ones_v[i, pl.ds(j * 16, 16)] = one16

  @pl.loop(0, RPS // ZROWS)
  def _(i):
    pltpu.sync_copy(zbuf, acc_sh.at[pl.ds(s * RPS + i * ZROWS, ZROWS)])

  pltpu.sync_copy(dst_hbm.at[wid], dst_v)
  plsc.subcore_barrier()

  @pl.loop(0, NCHUNK)
  def _(j):
    pltpu.sync_copy(ones_v, acc_sh.at[dst_v.at[j // GRP, j % GRP]], add=True)

  plsc.subcore_barrier()

  @pl.loop(0, RPS // ZROWS)
  def _(i):
    pltpu.sync_copy(acc_sh.at[pl.ds(s * RPS + i * ZROWS, ZROWS)], zbuf)
    pltpu.sync_copy(zbuf, deg_hbm.at[c].at[pl.ds(s * RPS + i * ZROWS, ZROWS)])


_sc_deg = pl.kernel(
    _sc_deg_body,
    out_type=jax.ShapeDtypeStruct((NC, NPAD, D), jnp.float32),
    mesh=_MESH,
    scratch_types=[
        pltpu.VMEM((NGRP, GRP, CHUNK), jnp.int32),
        pltpu.VMEM((CHUNK, D), jnp.float32),
        pltpu.VMEM((ZROWS, D), jnp.float32),
        pltpu.VMEM_SHARED((NPAD, D), jnp.float32),
    ])


def _tc_pre(x, w_self, w_neigh, b):
  def body(x_ref, ws_ref, wn_ref, b_ref, s_ref, g_ref):
    h = x_ref[...]
    s_ref[...] = jnp.dot(h, ws_ref[...],
                         preferred_element_type=jnp.float32) + b_ref[...]
    g_ref[...] = jnp.dot(h, wn_ref[...], preferred_element_type=jnp.float32)

  return pl.pallas_call(
      body,
      grid=(N // BR,),
      in_specs=[pl.BlockSpec((BR, D), lambda i: (i, 0)),
                pl.BlockSpec((D, D), lambda i: (0, 0)),
                pl.BlockSpec((D, D), lambda i: (0, 0)),
                pl.BlockSpec((1, D), lambda i: (0, 0))],
      out_specs=[pl.BlockSpec((BR, D), lambda i: (i, 0)),
                 pl.BlockSpec((BR, D), lambda i: (i, 0))],
      out_shape=[jax.ShapeDtypeStruct((N, D), jnp.float32)] * 2,
  )(x, w_self, w_neigh, b.reshape(1, D))


def _combine(s_ref, p_ref, deg_ref):
  p = p_ref[0] + p_ref[1]
  deg = deg_ref[0, :, 0:1] + deg_ref[1, :, 0:1]
  inv = 1.0 / jnp.maximum(deg, 1.0)
  return s_ref[...] + p * inv


def _tc_mid(s_in, p, degp, w_self, w_neigh, b):
  def body(s_ref, p_ref, deg_ref, ws_ref, wn_ref, b_ref, s_ref_o, g_ref_o):
    h = jnp.maximum(_combine(s_ref, p_ref, deg_ref), 0.0)
    s_ref_o[...] = jnp.dot(h, ws_ref[...],
                           preferred_element_type=jnp.float32) + b_ref[...]
    g_ref_o[...] = jnp.dot(h, wn_ref[...], preferred_element_type=jnp.float32)

  return pl.pallas_call(
      body,
      grid=(N // BR,),
      in_specs=[pl.BlockSpec((BR, D), lambda i: (i, 0)),
                pl.BlockSpec((NC, BR, D), lambda i: (0, i, 0)),
                pl.BlockSpec((NC, BR, D), lambda i: (0, i, 0)),
                pl.BlockSpec((D, D), lambda i: (0, 0)),
                pl.BlockSpec((D, D), lambda i: (0, 0)),
                pl.BlockSpec((1, D), lambda i: (0, 0))],
      out_specs=[pl.BlockSpec((BR, D), lambda i: (i, 0)),
                 pl.BlockSpec((BR, D), lambda i: (i, 0))],
      out_shape=[jax.ShapeDtypeStruct((N, D), jnp.float32)] * 2,
  )(s_in, p, degp, w_self, w_neigh, b.reshape(1, D))


def _tc_final(s_in, p, degp):
  def body(s_ref, p_ref, deg_ref, o_ref):
    o_ref[...] = _combine(s_ref, p_ref, deg_ref)

  return pl.pallas_call(
      body,
      grid=(N // BR,),
      in_specs=[pl.BlockSpec((BR, D), lambda i: (i, 0)),
                pl.BlockSpec((NC, BR, D), lambda i: (0, i, 0)),
                pl.BlockSpec((NC, BR, D), lambda i: (0, i, 0))],
      out_specs=pl.BlockSpec((BR, D), lambda i: (i, 0)),
      out_shape=jax.ShapeDtypeStruct((N, D), jnp.float32),
  )(s_in, p, degp)


def kernel(x, edge_index,
           W_self0, W_neigh0, b0,
           W_self1, W_neigh1, b1,
           W_self2, W_neigh2, b2,
           W_self3, W_neigh3, b3):
  ei = edge_index.astype(jnp.int32)
  pad_dst = N + jnp.arange(EPAD, dtype=jnp.int32) % (NPAD - N)
  pad_src = jnp.arange(EPAD, dtype=jnp.int32) % N
  src = jnp.concatenate([ei[0], pad_src])
  dst = jnp.concatenate([ei[1], pad_dst])
  src = src.reshape(NW, NGRP, GRP, CHUNK)
  dst = dst.reshape(NW, NGRP, GRP, CHUNK)

  degp = _sc_deg(dst)
  dep = degp[0, :8]
  s0, g0 = _tc_pre(x, W_self0, W_neigh0, b0)
  p0 = _sc_scatter(g0, src, dst, dep)
  s1, g1 = _tc_mid(s0, p0, degp, W_self1, W_neigh1, b1)
  p1 = _sc_scatter(g1, src, dst, dep)
  s2, g2 = _tc_mid(s1, p1, degp, W_self2, W_neigh2, b2)
  p2 = _sc_scatter(g2, src, dst, dep)
  s3, g3 = _tc_mid(s2, p2, degp, W_self3, W_neigh3, b3)
  p3 = _sc_scatter(g3, src, dst, dep)
  return _tc_final(s3, p3, degp)
